# Initial kernel scaffold; baseline (speedup 1.0000x reference)
import functools
import math

import jax
import jax.numpy as jnp
from jax import lax
from jax.experimental import pallas as pl
from jax.experimental.pallas import tpu as pltpu
from jax.experimental.pallas import tpu_sc as plsc

T = 2048
C = 768
H = 12
DH = C // H
E = 8
HID = int(C * 2.0)
EPS = 1e-8
DEPTH_SCALE = 1.0 / math.sqrt(1)

BLK = 256
NBLK = (2 * T) // BLK + (E - 1)
NPAD = NBLK * BLK

NC, NS = 2, 16
NW = NC * NS
TPW = T // NW
RPW = (2 * T) // NW

_F = jnp.float32
_I = jnp.int32


def _k1_body(x_ref, anw_ref, wq_ref, bq_ref, wkv_ref, bkv_ref,
             q_ref, k_ref, v_ref):
    x = x_ref[...]
    rms = jnp.sqrt(jnp.mean(x * x, axis=1, keepdims=True) + EPS)
    xn = x / rms * anw_ref[...][None, :]
    q = lax.dot_general(xn, wq_ref[...], (((1,), (1,)), ((), ())),
                        preferred_element_type=_F) + bq_ref[...][None, :]
    kv = lax.dot_general(xn, wkv_ref[...], (((1,), (1,)), ((), ())),
                         preferred_element_type=_F) + bkv_ref[...][None, :]
    q_ref[...] = q
    k_ref[...] = kv[:, :C]
    v_ref[...] = kv[:, C:]


def _run_k1(x2d, anw, wq, bq, wkv, bkv):
    return pl.pallas_call(
        _k1_body,
        out_shape=(jax.ShapeDtypeStruct((T, C), _F),
                   jax.ShapeDtypeStruct((T, C), _F),
                   jax.ShapeDtypeStruct((T, C), _F)),
    )(x2d, anw, wq, bq, wkv, bkv)


def _k2_body(q_ref, k_ref, v_ref, o_ref):
    h = pl.program_id(0)
    q = q_ref[...]
    k = k_ref[...]
    s = lax.dot_general(q, k, (((1,), (1,)), ((), ())),
                        preferred_element_type=_F) * (1.0 / math.sqrt(DH))
    slope = (h + 1).astype(_F) / H
    ii = lax.broadcasted_iota(_F, (T, T), 0)
    jj = lax.broadcasted_iota(_F, (T, T), 1)
    s = s + slope * (jj - ii)
    s = jnp.where(jj > ii, -1e30, s)
    m = jnp.max(s, axis=1, keepdims=True)
    p = jnp.exp(s - m)
    l = jnp.sum(p, axis=1, keepdims=True)
    o = lax.dot_general(p, v_ref[...], (((1,), (0,)), ((), ())),
                        preferred_element_type=_F)
    o_ref[...] = o / l


def _run_k2(q, k, v):
    spec = pl.BlockSpec((T, DH), lambda h: (0, h))
    return pl.pallas_call(
        _k2_body,
        grid=(H,),
        in_specs=[spec, spec, spec],
        out_specs=spec,
        out_shape=jax.ShapeDtypeStruct((T, C), _F),
    )(q, k, v)


def _k3_body(y_ref, x_ref, wo_ref, bo_ref, fw_ref, wr_ref,
             x2_ref, xn2_ref, pi_ref, wf_ref, ws_ref, be_ref):
    y = y_ref[...]
    o = lax.dot_general(y, wo_ref[...], (((1,), (1,)), ((), ())),
                        preferred_element_type=_F) + bo_ref[...][None, :]
    x2 = x_ref[...] + o
    x2_ref[...] = x2
    rms = jnp.sqrt(jnp.mean(x2 * x2, axis=1, keepdims=True) + EPS)
    xn2 = x2 / rms * fw_ref[...][None, :]
    xn2_ref[...] = xn2
    logits = lax.dot_general(xn2, wr_ref[...], (((1,), (1,)), ((), ())),
                             preferred_element_type=_F)
    io8 = lax.broadcasted_iota(_I, (T, E), 1)
    m1 = jnp.max(logits, axis=1, keepdims=True)
    idx1 = jnp.min(jnp.where(logits == m1, io8, E), axis=1, keepdims=True)
    l2 = jnp.where(io8 == idx1, -1e30, logits)
    m2 = jnp.max(l2, axis=1, keepdims=True)
    idx2 = jnp.min(jnp.where(l2 == m2, io8, E), axis=1, keepdims=True)
    t12 = jnp.exp(m2 - m1)
    p1 = 1.0 / (1.0 + t12)
    p2 = t12 / (1.0 + t12)
    oh = (io8 == idx1).astype(_F) + (io8 == idx2).astype(_F)
    tri = (lax.broadcasted_iota(_I, (T, T), 0)
           >= lax.broadcasted_iota(_I, (T, T), 1)).astype(_F)
    csum = lax.dot_general(tri, oh, (((1,), (0,)), ((), ())),
                           preferred_element_type=_F)
    tot = csum[T - 1:T, :]
    gb = jnp.floor((tot + (BLK - 1)) * (1.0 / BLK))
    u8 = (lax.broadcasted_iota(_I, (E, E), 0)
          < lax.broadcasted_iota(_I, (E, E), 1)).astype(_F)
    bo8 = lax.dot_general(gb, u8, (((1,), (0,)), ((), ())),
                          preferred_element_type=_F)
    pos8 = bo8 * float(BLK) + csum - 1.0
    ef = jnp.minimum(idx1, idx2)
    es = jnp.maximum(idx1, idx2)
    posf = jnp.sum((io8 == ef).astype(_F) * pos8, axis=1)
    poss = jnp.sum((io8 == es).astype(_F) * pos8, axis=1)
    pi_ref[pl.ds(0, T)] = posf.astype(_I)
    pi_ref[pl.ds(T, T)] = poss.astype(_I)
    wf_ref[...] = jnp.where(idx1 < idx2, p1, p2) * DEPTH_SCALE
    ws_ref[...] = jnp.where(idx1 < idx2, p2, p1) * DEPTH_SCALE
    bvec = lax.broadcasted_iota(_F, (NBLK, E), 0)
    be = jnp.sum((bo8 <= bvec).astype(_F), axis=1) - 1.0
    be_ref[...] = be.astype(_I)


def _run_k3(y, x2d, wo, bo, fw, wr):
    return pl.pallas_call(
        _k3_body,
        out_shape=(jax.ShapeDtypeStruct((T, C), _F),
                   jax.ShapeDtypeStruct((T, C), _F),
                   jax.ShapeDtypeStruct((2 * T,), _I),
                   jax.ShapeDtypeStruct((T, 1), _F),
                   jax.ShapeDtypeStruct((T, 1), _F),
                   jax.ShapeDtypeStruct((NBLK,), _I)),
    )(y, x2d, wo, bo, fw, wr)


def _sc_mesh():
    return plsc.VectorSubcoreMesh(core_axis_name="c", subcore_axis_name="s")


def _sc_dispatch_body(xn2_hbm, pi_hbm, xs_hbm, rows_v, i0_v, i1_v, sem):
    wid = lax.axis_index("s") * NC + lax.axis_index("c")
    base = wid * TPW
    pltpu.sync_copy(xn2_hbm.at[pl.ds(base, TPW)], rows_v)
    pltpu.sync_copy(pi_hbm.at[pl.ds(base, TPW)], i0_v)
    pltpu.sync_copy(pi_hbm.at[pl.ds(T + base, TPW)], i1_v)
    c0 = pltpu.async_copy(rows_v, xs_hbm.at[i0_v], sem)
    c1 = pltpu.async_copy(rows_v, xs_hbm.at[i1_v], sem)
    c0.wait()
    c1.wait()


def _dispatch_sc(xn2, pi):
    run = functools.partial(
        pl.kernel,
        mesh=_sc_mesh(),
        out_type=jax.ShapeDtypeStruct((NPAD, C), _F),
        scratch_types=[pltpu.VMEM((TPW, C), _F),
                       pltpu.VMEM((TPW,), _I),
                       pltpu.VMEM((TPW,), _I),
                       pltpu.SemaphoreType.DMA],
    )(_sc_dispatch_body)
    return run(xn2, pi)


def _k4_body(be_ref, xs_ref, wg_ref, wu_ref, wd_ref, o_ref):
    xs = xs_ref[...]
    g = lax.dot_general(xs, wg_ref[0], (((1,), (1,)), ((), ())),
                        preferred_element_type=_F)
    u = lax.dot_general(xs, wu_ref[0], (((1,), (1,)), ((), ())),
                        preferred_element_type=_F)
    hdn = (g / (1.0 + jnp.exp(-g))) * u
    o_ref[...] = lax.dot_general(hdn, wd_ref[0], (((1,), (1,)), ((), ())),
                                 preferred_element_type=_F)


def _run_k4(bexp, xs, wg, wu, wd):
    grid_spec = pltpu.PrefetchScalarGridSpec(
        num_scalar_prefetch=1,
        grid=(NBLK,),
        in_specs=[
            pl.BlockSpec((BLK, C), lambda b, be: (b, 0)),
            pl.BlockSpec((1, HID, C), lambda b, be: (be[b], 0, 0)),
            pl.BlockSpec((1, HID, C), lambda b, be: (be[b], 0, 0)),
            pl.BlockSpec((1, C, HID), lambda b, be: (be[b], 0, 0)),
        ],
        out_specs=pl.BlockSpec((BLK, C), lambda b, be: (b, 0)),
    )
    return pl.pallas_call(
        _k4_body,
        grid_spec=grid_spec,
        out_shape=jax.ShapeDtypeStruct((NPAD, C), _F),
    )(bexp, xs, wg, wu, wd)


def _sc_combine_body(outs_hbm, pi_hbm, g_hbm, i_v, rows_v, sem):
    wid = lax.axis_index("s") * NC + lax.axis_index("c")
    base = wid * RPW
    pltpu.sync_copy(pi_hbm.at[pl.ds(base, RPW)], i_v)
    pltpu.async_copy(outs_hbm.at[i_v], rows_v, sem).wait()
    pltpu.sync_copy(rows_v, g_hbm.at[pl.ds(base, RPW)])


def _combine_sc(outs, pi):
    run = functools.partial(
        pl.kernel,
        mesh=_sc_mesh(),
        out_type=jax.ShapeDtypeStruct((2 * T, C), _F),
        scratch_types=[pltpu.VMEM((RPW,), _I),
                       pltpu.VMEM((RPW, C), _F),
                       pltpu.SemaphoreType.DMA],
    )(_sc_combine_body)
    return run(outs, pi)


def _k5_body(x2_ref, g0_ref, g1_ref, wf_ref, ws_ref, o_ref):
    o_ref[...] = (x2_ref[...]
                  + wf_ref[...] * g0_ref[...]
                  + ws_ref[...] * g1_ref[...])


def _run_k5(x2, g, wf, ws):
    nb = 8
    blk = T // nb
    return pl.pallas_call(
        _k5_body,
        grid=(nb,),
        in_specs=[
            pl.BlockSpec((blk, C), lambda t: (t, 0)),
            pl.BlockSpec((blk, C), lambda t: (t, 0)),
            pl.BlockSpec((blk, C), lambda t: (t + nb, 0)),
            pl.BlockSpec((blk, 1), lambda t: (t, 0)),
            pl.BlockSpec((blk, 1), lambda t: (t, 0)),
        ],
        out_specs=pl.BlockSpec((blk, C), lambda t: (t, 0)),
        out_shape=jax.ShapeDtypeStruct((T, C), _F),
    )(x2, g, g, wf, ws)


def kernel(x, attn_norm_w, Wq, bq, Wkv, bkv, Wo, bo, ffn_norm_w,
           Wr, Wg, Wu, Wd):
    x2d = x.reshape(T, C)
    q, k, v = _run_k1(x2d, attn_norm_w, Wq, bq, Wkv, bkv)
    y = _run_k2(q, k, v)
    x2, xn2, pi, wf, ws, bexp = _run_k3(y, x2d, Wo, bo, ffn_norm_w, Wr)
    xs = _dispatch_sc(xn2, pi)
    outs = _run_k4(bexp, xs, Wg, Wu, Wd)
    g = _combine_sc(outs, pi)
    out = _run_k5(x2, g, wf, ws)
    return out.reshape(1, T, C)


# speedup vs baseline: 1.5628x; 1.5628x over previous
import functools
import math

import jax
import jax.numpy as jnp
from jax import lax
from jax.experimental import pallas as pl
from jax.experimental.pallas import tpu as pltpu
from jax.experimental.pallas import tpu_sc as plsc

T = 2048
C = 768
H = 12
DH = C // H
E = 8
HID = int(C * 2.0)
EPS = 1e-8
DEPTH_SCALE = 1.0 / math.sqrt(1)

BLK = 256
NBLK = (2 * T) // BLK + (E - 1)
NPAD = NBLK * BLK

NC, NS = 2, 16
NW = NC * NS
TPW = T // NW
RPW = (2 * T) // NW

_F = jnp.float32
_I = jnp.int32


def _k1_body(x_ref, anw_ref, wq_ref, bq_ref, wkv_ref, bkv_ref,
             q_ref, k_ref, v_ref):
    x = x_ref[...]
    rms = jnp.sqrt(jnp.mean(x * x, axis=1, keepdims=True) + EPS)
    xn = x / rms * anw_ref[...][None, :]
    q = lax.dot_general(xn, wq_ref[...], (((1,), (1,)), ((), ())),
                        preferred_element_type=_F) + bq_ref[...][None, :]
    kv = lax.dot_general(xn, wkv_ref[...], (((1,), (1,)), ((), ())),
                         preferred_element_type=_F) + bkv_ref[...][None, :]
    tb = q.shape[0]
    q_ref[...] = q.reshape(tb, H, DH).transpose(1, 0, 2)
    k_ref[...] = kv[:, :C].reshape(tb, H, DH).transpose(1, 0, 2)
    v_ref[...] = kv[:, C:].reshape(tb, H, DH).transpose(1, 0, 2)


_K1_TB = 512


def _run_k1(x2d, anw, wq, bq, wkv, bkv):
    xspec = pl.BlockSpec((_K1_TB, C), lambda t: (t, 0))
    full = lambda shape: pl.BlockSpec(shape, lambda t: tuple(0 for _ in shape))
    ospec = pl.BlockSpec((H, _K1_TB, DH), lambda t: (0, t, 0))
    return pl.pallas_call(
        _k1_body,
        grid=(T // _K1_TB,),
        in_specs=[xspec, full((C,)), full((C, C)), full((C,)),
                  full((2 * C, C)), full((2 * C,))],
        out_specs=(ospec, ospec, ospec),
        out_shape=(jax.ShapeDtypeStruct((H, T, DH), _F),
                   jax.ShapeDtypeStruct((H, T, DH), _F),
                   jax.ShapeDtypeStruct((H, T, DH), _F)),
    )(x2d, anw, wq, bq, wkv, bkv)


_K2_TQ = 512


def _k2_body(q_ref, k_ref, v_ref, o_ref):
    h = pl.program_id(0)
    tq = pl.program_id(1)
    q = q_ref[0]
    k = k_ref[0]
    s = lax.dot_general(q, k, (((1,), (1,)), ((), ())),
                        preferred_element_type=_F) * (1.0 / math.sqrt(DH))
    slope = (h + 1).astype(_F) / H
    icol = (lax.broadcasted_iota(_I, (_K2_TQ, 1), 0)
            + tq * _K2_TQ).astype(_F)
    jrow = lax.broadcasted_iota(_I, (1, T), 1).astype(_F)
    d = jrow - icol
    s = s + slope * d + jnp.where(d > 0.0, -1e30, 0.0)
    m = jnp.max(s, axis=1, keepdims=True)
    p = jnp.exp(s - m)
    l = jnp.sum(p, axis=1, keepdims=True)
    o = lax.dot_general(p, v_ref[0], (((1,), (0,)), ((), ())),
                        preferred_element_type=_F)
    o_ref[0] = o / l


def _run_k2(q, k, v):
    qspec = pl.BlockSpec((1, _K2_TQ, DH), lambda h, tq: (h, tq, 0))
    kspec = pl.BlockSpec((1, T, DH), lambda h, tq: (h, 0, 0))
    return pl.pallas_call(
        _k2_body,
        grid=(H, T // _K2_TQ),
        in_specs=[qspec, kspec, kspec],
        out_specs=qspec,
        out_shape=jax.ShapeDtypeStruct((H, T, DH), _F),
    )(q, k, v)


def _k3_body(y_ref, x_ref, wo_ref, bo_ref, fw_ref, wr_ref,
             x2_ref, xn2_ref, pi_ref, wf_ref, ws_ref, be_ref):
    y = y_ref[...].transpose(1, 0, 2).reshape(T, C)
    o = lax.dot_general(y, wo_ref[...], (((1,), (1,)), ((), ())),
                        preferred_element_type=_F) + bo_ref[...][None, :]
    x2 = x_ref[...] + o
    x2_ref[...] = x2
    rms = jnp.sqrt(jnp.mean(x2 * x2, axis=1, keepdims=True) + EPS)
    xn2 = x2 / rms * fw_ref[...][None, :]
    xn2_ref[...] = xn2
    logits = lax.dot_general(xn2, wr_ref[...], (((1,), (1,)), ((), ())),
                             preferred_element_type=_F)
    io8 = lax.broadcasted_iota(_I, (T, E), 1)
    m1 = jnp.max(logits, axis=1, keepdims=True)
    idx1 = jnp.min(jnp.where(logits == m1, io8, E), axis=1, keepdims=True)
    l2 = jnp.where(io8 == idx1, -1e30, logits)
    m2 = jnp.max(l2, axis=1, keepdims=True)
    idx2 = jnp.min(jnp.where(l2 == m2, io8, E), axis=1, keepdims=True)
    t12 = jnp.exp(m2 - m1)
    p1 = 1.0 / (1.0 + t12)
    p2 = t12 / (1.0 + t12)
    oh = (io8 == idx1).astype(_F) + (io8 == idx2).astype(_F)
    cb = 128
    tri = (lax.broadcasted_iota(_I, (cb, cb), 0)
           >= lax.broadcasted_iota(_I, (cb, cb), 1)).astype(_F)
    chunks = []
    carry = jnp.zeros((1, E), _F)
    for i in range(T // cb):
        c = lax.dot_general(tri, oh[i * cb:(i + 1) * cb, :],
                            (((1,), (0,)), ((), ())),
                            preferred_element_type=_F) + carry
        chunks.append(c)
        carry = c[cb - 1:cb, :]
    csum = jnp.concatenate(chunks, axis=0)
    tot = carry
    gb = jnp.floor((tot + (BLK - 1)) * (1.0 / BLK))
    u8 = (lax.broadcasted_iota(_I, (E, E), 0)
          < lax.broadcasted_iota(_I, (E, E), 1)).astype(_F)
    bo8 = lax.dot_general(gb, u8, (((1,), (0,)), ((), ())),
                          preferred_element_type=_F)
    pos8 = bo8 * float(BLK) + csum - 1.0
    ef = jnp.minimum(idx1, idx2)
    es = jnp.maximum(idx1, idx2)
    posf = jnp.sum((io8 == ef).astype(_F) * pos8, axis=1)
    poss = jnp.sum((io8 == es).astype(_F) * pos8, axis=1)
    pi_ref[pl.ds(0, T)] = posf.astype(_I)
    pi_ref[pl.ds(T, T)] = poss.astype(_I)
    wf_ref[...] = jnp.where(idx1 < idx2, p1, p2) * DEPTH_SCALE
    ws_ref[...] = jnp.where(idx1 < idx2, p2, p1) * DEPTH_SCALE
    bvec = lax.broadcasted_iota(_I, (NBLK, E), 0).astype(_F)
    be = jnp.sum((bo8 <= bvec).astype(_F), axis=1) - 1.0
    be_ref[...] = be.astype(_I)


def _run_k3(y, x2d, wo, bo, fw, wr):
    return pl.pallas_call(
        _k3_body,
        out_shape=(jax.ShapeDtypeStruct((T, C), _F),
                   jax.ShapeDtypeStruct((T, C), _F),
                   jax.ShapeDtypeStruct((2 * T,), _I),
                   jax.ShapeDtypeStruct((T, 1), _F),
                   jax.ShapeDtypeStruct((T, 1), _F),
                   jax.ShapeDtypeStruct((NBLK,), _I)),
    )(y, x2d, wo, bo, fw, wr)


def _sc_mesh():
    return plsc.VectorSubcoreMesh(core_axis_name="c", subcore_axis_name="s")


def _sc_dispatch_body(xn2_hbm, pi_hbm, xs_hbm, rows_v, i0_v, i1_v, sem):
    wid = lax.axis_index("s") * NC + lax.axis_index("c")
    base = wid * TPW
    pltpu.sync_copy(xn2_hbm.at[pl.ds(base, TPW)], rows_v)
    pltpu.sync_copy(pi_hbm.at[pl.ds(base, TPW)], i0_v)
    pltpu.sync_copy(pi_hbm.at[pl.ds(T + base, TPW)], i1_v)
    c0 = pltpu.async_copy(rows_v, xs_hbm.at[i0_v], sem)
    c1 = pltpu.async_copy(rows_v, xs_hbm.at[i1_v], sem)
    c0.wait()
    c1.wait()


def _dispatch_sc(xn2, pi):
    run = functools.partial(
        pl.kernel,
        mesh=_sc_mesh(),
        out_type=jax.ShapeDtypeStruct((NPAD, C), _F),
        scratch_types=[pltpu.VMEM((TPW, C), _F),
                       pltpu.VMEM((TPW,), _I),
                       pltpu.VMEM((TPW,), _I),
                       pltpu.SemaphoreType.DMA],
    )(_sc_dispatch_body)
    return run(xn2, pi)


def _k4_body(be_ref, xs_ref, wg_ref, wu_ref, wd_ref, o_ref):
    xs = xs_ref[...]
    g = lax.dot_general(xs, wg_ref[0], (((1,), (1,)), ((), ())),
                        preferred_element_type=_F)
    u = lax.dot_general(xs, wu_ref[0], (((1,), (1,)), ((), ())),
                        preferred_element_type=_F)
    hdn = (g / (1.0 + jnp.exp(-g))) * u
    o_ref[...] = lax.dot_general(hdn, wd_ref[0], (((1,), (1,)), ((), ())),
                                 preferred_element_type=_F)


def _run_k4(bexp, xs, wg, wu, wd):
    grid_spec = pltpu.PrefetchScalarGridSpec(
        num_scalar_prefetch=1,
        grid=(NBLK,),
        in_specs=[
            pl.BlockSpec((BLK, C), lambda b, be: (b, 0)),
            pl.BlockSpec((1, HID, C), lambda b, be: (be[b], 0, 0)),
            pl.BlockSpec((1, HID, C), lambda b, be: (be[b], 0, 0)),
            pl.BlockSpec((1, C, HID), lambda b, be: (be[b], 0, 0)),
        ],
        out_specs=pl.BlockSpec((BLK, C), lambda b, be: (b, 0)),
    )
    return pl.pallas_call(
        _k4_body,
        grid_spec=grid_spec,
        out_shape=jax.ShapeDtypeStruct((NPAD, C), _F),
    )(bexp, xs, wg, wu, wd)


def _sc_combine_body(outs_hbm, pi_hbm, g_hbm, i_v, rows_v, sem):
    wid = lax.axis_index("s") * NC + lax.axis_index("c")
    base = wid * RPW
    pltpu.sync_copy(pi_hbm.at[pl.ds(base, RPW)], i_v)
    pltpu.async_copy(outs_hbm.at[i_v], rows_v, sem).wait()
    pltpu.sync_copy(rows_v, g_hbm.at[pl.ds(base, RPW)])


def _combine_sc(outs, pi):
    run = functools.partial(
        pl.kernel,
        mesh=_sc_mesh(),
        out_type=jax.ShapeDtypeStruct((2 * T, C), _F),
        scratch_types=[pltpu.VMEM((RPW,), _I),
                       pltpu.VMEM((RPW, C), _F),
                       pltpu.SemaphoreType.DMA],
    )(_sc_combine_body)
    return run(outs, pi)


def _k5_body(x2_ref, g0_ref, g1_ref, wf_ref, ws_ref, o_ref):
    o_ref[...] = (x2_ref[...]
                  + wf_ref[...] * g0_ref[...]
                  + ws_ref[...] * g1_ref[...])


def _run_k5(x2, g, wf, ws):
    nb = 8
    blk = T // nb
    return pl.pallas_call(
        _k5_body,
        grid=(nb,),
        in_specs=[
            pl.BlockSpec((blk, C), lambda t: (t, 0)),
            pl.BlockSpec((blk, C), lambda t: (t, 0)),
            pl.BlockSpec((blk, C), lambda t: (t + nb, 0)),
            pl.BlockSpec((blk, 1), lambda t: (t, 0)),
            pl.BlockSpec((blk, 1), lambda t: (t, 0)),
        ],
        out_specs=pl.BlockSpec((blk, C), lambda t: (t, 0)),
        out_shape=jax.ShapeDtypeStruct((T, C), _F),
    )(x2, g, g, wf, ws)


def kernel(x, attn_norm_w, Wq, bq, Wkv, bkv, Wo, bo, ffn_norm_w,
           Wr, Wg, Wu, Wd):
    x2d = x.reshape(T, C)
    q, k, v = _run_k1(x2d, attn_norm_w, Wq, bq, Wkv, bkv)
    y = _run_k2(q, k, v)
    x2, xn2, pi, wf, ws, bexp = _run_k3(y, x2d, Wo, bo, ffn_norm_w, Wr)
    xs = _dispatch_sc(xn2, pi)
    outs = _run_k4(bexp, xs, Wg, Wu, Wd)
    g = _combine_sc(outs, pi)
    out = _run_k5(x2, g, wf, ws)
    return out.reshape(1, T, C)
